# baseline (speedup 1.0000x reference)
import functools

import jax
import jax.numpy as jnp
from jax import lax
from jax.experimental import pallas as pl
from jax.experimental.pallas import tpu as pltpu
from jax.experimental.pallas import tpu_sc as plsc

_INFO = plsc.get_sparse_core_info()
_NC, _NS, _L = _INFO.num_cores, _INFO.num_subcores, _INFO.num_lanes

_B = 512
_NBUF = 2


@functools.partial(jax.jit, static_argnums=(2, 3, 4))
def _col2octree_sc(data_t, octree_t, C, K, H):
    n_chunks = H // _B

    mesh = plsc.VectorSubcoreMesh(core_axis_name="c", subcore_axis_name="s")

    @functools.partial(
        pl.kernel,
        mesh=mesh,
        out_type=jax.ShapeDtypeStruct((C, H), jnp.float32),
        compiler_params=pltpu.CompilerParams(needs_layout_passes=False),
        scratch_types=[
            pltpu.VMEM((H,), jnp.float32),
            pltpu.VMEM((K, _B), jnp.int32),
            pltpu.VMEM((K, _B), jnp.int32),
            pltpu.VMEM((K, _B), jnp.float32),
            pltpu.VMEM((K, _B), jnp.float32),
            pltpu.SemaphoreType.DMA,
            pltpu.SemaphoreType.DMA,
            pltpu.SemaphoreType.DMA,
            pltpu.SemaphoreType.DMA,
        ],
    )
    def k(data_hbm, idx_hbm, out_hbm, accum, idxb0, idxb1, datab0, datab1,
          si0, si1, sd0, sd1):
        ch = lax.axis_index("s") * _NC + lax.axis_index("c")
        idxbs = (idxb0, idxb1)
        databs = (datab0, datab1)
        sems_i = (si0, si1)
        sems_d = (sd0, sd1)

        def start(g, b):
            pltpu.async_copy(
                idx_hbm.at[:, pl.ds(g * _B, _B)], idxbs[b], sems_i[b]
            )
            pltpu.async_copy(
                data_hbm.at[:, ch, pl.ds(g * _B, _B)], databs[b], sems_d[b]
            )

        def wait(g, b):
            pltpu.make_async_copy(
                idx_hbm.at[:, pl.ds(g * _B, _B)], idxbs[b], sems_i[b]
            ).wait()
            pltpu.make_async_copy(
                data_hbm.at[:, ch, pl.ds(g * _B, _B)], databs[b], sems_d[b]
            ).wait()

        start(0, 0)
        start(1, 1)

        zeros = jnp.zeros((_L,), jnp.float32)

        def zbody(i, carry):
            accum[pl.ds(i * _L, _L)] = zeros
            return carry

        lax.fori_loop(0, H // _L, zbody, 0)

        def outer(gg, carry):
            for b in range(_NBUF):
                g = gg * _NBUF + b
                wait(g, b)
                idxb = idxbs[b]
                datab = databs[b]

                def ibody(i, icarry):
                    for u in range(2):
                        sl = pl.ds((i * 2 + u) * _L, _L)
                        for kb in range(0, K, 9):
                            kks = range(kb, min(kb + 9, K))
                            vis = [idxb[kk, sl] for kk in kks]
                            vds = [datab[kk, sl] for kk in kks]
                            for vi, vd in zip(vis, vds):
                                plsc.addupdate_scatter(accum, [vi], vd)
                    return icarry

                lax.fori_loop(0, _B // _L // 2, ibody, 0)

                @pl.when(g + _NBUF < n_chunks)
                def _():
                    start(g + _NBUF, b)

            return carry

        lax.fori_loop(0, n_chunks // _NBUF, outer, 0)
        pltpu.sync_copy(accum, out_hbm.at[ch])

    return k(data_t, octree_t)


def kernel(data_in, octree):
    C, K, H = data_in.shape
    data_t = jnp.transpose(data_in, (1, 0, 2))
    octree_t = octree.T
    return _col2octree_sc(data_t, octree_t, C, K, H)


# speedup vs baseline: 1.0011x; 1.0011x over previous
import functools

import jax
import jax.numpy as jnp
from jax import lax
from jax.experimental import pallas as pl
from jax.experimental.pallas import tpu as pltpu
from jax.experimental.pallas import tpu_sc as plsc

_INFO = plsc.get_sparse_core_info()
_NC, _NS, _L = _INFO.num_cores, _INFO.num_subcores, _INFO.num_lanes

_B = 512
_NBUF = 2


@functools.partial(jax.jit, static_argnums=(2, 3, 4))
def _col2octree_sc(data_t, octree_t, C, K, H):
    n_chunks = H // _B

    mesh = plsc.VectorSubcoreMesh(core_axis_name="c", subcore_axis_name="s")

    @functools.partial(
        pl.kernel,
        mesh=mesh,
        out_type=jax.ShapeDtypeStruct((C, H), jnp.float32),
        compiler_params=pltpu.CompilerParams(needs_layout_passes=False),
        scratch_types=[
            pltpu.VMEM((H,), jnp.float32),
            pltpu.VMEM((K, _B), jnp.int32),
            pltpu.VMEM((K, _B), jnp.int32),
            pltpu.VMEM((K, _B), jnp.float32),
            pltpu.VMEM((K, _B), jnp.float32),
            pltpu.SemaphoreType.DMA,
            pltpu.SemaphoreType.DMA,
            pltpu.SemaphoreType.DMA,
            pltpu.SemaphoreType.DMA,
        ],
    )
    def k(data_hbm, idx_hbm, out_hbm, accum, idxb0, idxb1, datab0, datab1,
          si0, si1, sd0, sd1):
        ch = lax.axis_index("s") * _NC + lax.axis_index("c")
        idxbs = (idxb0, idxb1)
        databs = (datab0, datab1)
        sems_i = (si0, si1)
        sems_d = (sd0, sd1)

        def start(g, b):
            pltpu.async_copy(
                idx_hbm.at[:, pl.ds(g * _B, _B)], idxbs[b], sems_i[b]
            )
            pltpu.async_copy(
                data_hbm.at[:, ch, pl.ds(g * _B, _B)], databs[b], sems_d[b]
            )

        def wait(g, b):
            pltpu.make_async_copy(
                idx_hbm.at[:, pl.ds(g * _B, _B)], idxbs[b], sems_i[b]
            ).wait()
            pltpu.make_async_copy(
                data_hbm.at[:, ch, pl.ds(g * _B, _B)], databs[b], sems_d[b]
            ).wait()

        start(0, 0)
        start(1, 1)

        zeros = jnp.zeros((_L,), jnp.float32)

        def zbody(i, carry):
            accum[pl.ds(i * _L, _L)] = zeros
            return carry

        lax.fori_loop(0, H // _L, zbody, 0)

        def outer(gg, carry):
            for b in range(_NBUF):
                g = gg * _NBUF + b
                wait(g, b)
                idxb = idxbs[b]
                datab = databs[b]

                def ibody(i, icarry):
                    sl = pl.ds(i * _L, _L)
                    for kb in range(0, K, 9):
                        kks = range(kb, min(kb + 9, K))
                        vis = [idxb[kk, sl] for kk in kks]
                        vds = [datab[kk, sl] for kk in kks]
                        for vi, vd in zip(vis, vds):
                            plsc.addupdate_scatter(accum, [vi], vd)
                    return icarry

                lax.fori_loop(0, _B // _L, ibody, 0)

                @pl.when(g + _NBUF < n_chunks)
                def _():
                    start(g + _NBUF, b)

            return carry

        lax.fori_loop(0, n_chunks // _NBUF, outer, 0)
        pltpu.sync_copy(accum, out_hbm.at[ch])

    return k(data_t, octree_t)


def kernel(data_in, octree):
    C, K, H = data_in.shape
    data_t = jnp.transpose(data_in, (1, 0, 2))
    octree_t = octree.T
    return _col2octree_sc(data_t, octree_t, C, K, H)
